# Initial kernel scaffold; baseline (speedup 1.0000x reference)
import jax
import jax.numpy as jnp
from jax.experimental import pallas as pl


def kernel(x, edge_index, A0, A1, ws_weights, W_l, b_l, W_r):
    raise NotImplementedError("write your pallas kernel here")


# speedup vs baseline: 2.6858x; 2.6858x over previous
import functools

import jax
import jax.numpy as jnp
from jax import lax
from jax.experimental import pallas as pl
from jax.experimental.pallas import tpu as pltpu
from jax.experimental.pallas import tpu_sc as plsc

N = 10000
D = 128
E = 320000

NC = 2
NS = 16
NW = NC * NS
K = 128
PER_W = 10240
CHUNKS = PER_W // K
E_PAD = NW * PER_W
NP = 10240
ROWS_PER_TILE = NP // NS
DUMMY = N
ZR = 8


def _sc_scatter(src_p, dstr_p, a0_p, a1_p, wvec, x):
    mesh = plsc.VectorSubcoreMesh(
        core_axis_name="c", subcore_axis_name="s", num_cores=NC, num_subcores=NS
    )

    @functools.partial(
        pl.kernel,
        out_type=jax.ShapeDtypeStruct((NC * NP, D), jnp.float32),
        mesh=mesh,
        scratch_types=[
            pltpu.VMEM((K,), jnp.int32),
            pltpu.VMEM((K,), jnp.int32),
            pltpu.VMEM((K,), jnp.float32),
            pltpu.VMEM((K,), jnp.float32),
            pltpu.VMEM((2, 16), jnp.float32),
            pltpu.VMEM((K, D), jnp.float32),
            pltpu.VMEM((ZR, D), jnp.float32),
            pltpu.VMEM_SHARED((NP, D), jnp.float32),
            pltpu.SemaphoreType.DMA,
        ],
    )
    def sc_kernel(src_hbm, dst_hbm, a0_hbm, a1_hbm, w_hbm, x_hbm, out_hbm,
                  src_v, dst_v, a0_v, a1_v, w_v, rows_v, z_v, acc, sem):
        cid = lax.axis_index("c")
        sid = lax.axis_index("s")
        wid = sid * NC + cid

        zeros16 = jnp.zeros((16,), jnp.float32)
        for r in range(ZR):
            for c in range(D // 16):
                z_v[r, pl.ds(c * 16, 16)] = zeros16

        def zero_body(j, carry):
            pltpu.sync_copy(z_v, acc.at[pl.ds(sid * ROWS_PER_TILE + j * ZR, ZR)])
            return carry

        lax.fori_loop(0, ROWS_PER_TILE // ZR, zero_body, 0)
        plsc.subcore_barrier()

        pltpu.sync_copy(w_hbm, w_v)
        w0 = w_v[0]
        w1 = w_v[1]

        base = wid * PER_W

        def chunk_body(i, carry):
            off = base + i * K
            pltpu.sync_copy(src_hbm.at[pl.ds(off, K)], src_v)
            pltpu.sync_copy(dst_hbm.at[pl.ds(off, K)], dst_v)
            pltpu.sync_copy(a0_hbm.at[pl.ds(off, K)], a0_v)
            pltpu.sync_copy(a1_hbm.at[pl.ds(off, K)], a1_v)
            dummy16 = jnp.full((16,), DUMMY, jnp.int32)
            for v in range(K // 16):
                sl = pl.ds(v * 16, 16)
                ew = w0 * a0_v[sl] + w1 * a1_v[sl]
                m = ew != 0.0
                dst_v[sl] = jnp.where(m, dst_v[sl], dummy16)
            pltpu.async_copy(x_hbm.at[src_v], rows_v, sem).wait()
            pltpu.sync_copy(rows_v, acc.at[dst_v], add=True)
            return carry

        lax.fori_loop(0, CHUNKS, chunk_body, 0)
        plsc.subcore_barrier()

        pltpu.sync_copy(
            acc.at[pl.ds(sid * ROWS_PER_TILE, ROWS_PER_TILE)],
            out_hbm.at[pl.ds(cid * NP + sid * ROWS_PER_TILE, ROWS_PER_TILE)],
        )

    return sc_kernel(src_p, dstr_p, a0_p, a1_p, wvec, x)


def _tc_body(p0_ref, p1_ref, x_ref, wl_ref, wr_ref, b_ref, o_ref):
    agg = p0_ref[...] + p1_ref[...]
    dn = (((1,), (1,)), ((), ()))
    o_ref[...] = (
        lax.dot_general(agg, wl_ref[...], dn, preferred_element_type=jnp.float32)
        + lax.dot_general(x_ref[...], wr_ref[...], dn, preferred_element_type=jnp.float32)
        + b_ref[0:1, :]
    )


def _tc_dense(p0, p1, x, W_l, W_r, b8):
    blk = 1000
    grid = (N // blk,)
    return pl.pallas_call(
        _tc_body,
        grid=grid,
        in_specs=[
            pl.BlockSpec((blk, D), lambda i: (i, 0)),
            pl.BlockSpec((blk, D), lambda i: (i, 0)),
            pl.BlockSpec((blk, D), lambda i: (i, 0)),
            pl.BlockSpec((D, D), lambda i: (0, 0)),
            pl.BlockSpec((D, D), lambda i: (0, 0)),
            pl.BlockSpec((8, D), lambda i: (0, 0)),
        ],
        out_specs=pl.BlockSpec((blk, D), lambda i: (i, 0)),
        out_shape=jax.ShapeDtypeStruct((N, D), jnp.float32),
    )(p0, p1, x, W_l, W_r, b8)


def kernel(x, edge_index, A0, A1, ws_weights, W_l, b_l, W_r):
    src = edge_index[0].astype(jnp.int32)
    dst = edge_index[1].astype(jnp.int32)
    pad = E_PAD - E
    src_p = jnp.concatenate([src, jnp.zeros((pad,), jnp.int32)])
    dst_p = jnp.concatenate([dst, jnp.zeros((pad,), jnp.int32)])
    a0_p = jnp.concatenate([A0, jnp.zeros((pad,), jnp.float32)])
    a1_p = jnp.concatenate([A1, jnp.zeros((pad,), jnp.float32)])
    wvec = jnp.stack([
        jnp.full((16,), ws_weights[0], jnp.float32),
        jnp.full((16,), ws_weights[1], jnp.float32),
    ])

    partials = _sc_scatter(src_p, dst_p, a0_p, a1_p, wvec, x)
    p0 = partials[:N]
    p1 = partials[NP:NP + N]
    b8 = jnp.broadcast_to(b_l.reshape(1, D), (8, D))
    return _tc_dense(p0, p1, x, W_l, W_r, b8)


# speedup vs baseline: 3.6148x; 1.3459x over previous
import functools

import jax
import jax.numpy as jnp
from jax import lax
from jax.experimental import pallas as pl
from jax.experimental.pallas import tpu as pltpu
from jax.experimental.pallas import tpu_sc as plsc

N = 10000
D = 128
E = 320000

NC = 2
NS = 16
NW = NC * NS
K = 128
PER_W = 10240
CHUNKS = PER_W // K
E_PAD = NW * PER_W
NP = 10240
ROWS_PER_TILE = NP // NS
DUMMY = N
ZR = 8


def _sc_scatter(edata_i, edata_f, wvec, x):
    mesh = plsc.VectorSubcoreMesh(
        core_axis_name="c", subcore_axis_name="s", num_cores=NC, num_subcores=NS
    )

    @functools.partial(
        pl.kernel,
        out_type=jax.ShapeDtypeStruct((NC * NP, D), jnp.float32),
        mesh=mesh,
        scratch_types=[
            pltpu.VMEM((2 * K,), jnp.int32),
            pltpu.VMEM((2 * K,), jnp.int32),
            pltpu.VMEM((2 * K,), jnp.float32),
            pltpu.VMEM((2 * K,), jnp.float32),
            pltpu.VMEM((2, 16), jnp.float32),
            pltpu.VMEM((K, D), jnp.float32),
            pltpu.VMEM((K, D), jnp.float32),
            pltpu.VMEM((K,), jnp.int32),
            pltpu.VMEM((K,), jnp.int32),
            pltpu.VMEM((ZR, D), jnp.float32),
            pltpu.VMEM_SHARED((NP, D), jnp.float32),
            pltpu.SemaphoreType.DMA,
            pltpu.SemaphoreType.DMA,
            pltpu.SemaphoreType.DMA,
            pltpu.SemaphoreType.DMA,
            pltpu.SemaphoreType.DMA,
            pltpu.SemaphoreType.DMA,
        ],
    )
    def sc_kernel(edi_hbm, edf_hbm, w_hbm, x_hbm, out_hbm,
                  ed0, ed1, ef0, ef1, w_v, rows0, rows1, idx0, idx1,
                  z_v, acc, esem0, esem1, fsem0, fsem1, gsem0, gsem1):
        cid = lax.axis_index("c")
        sid = lax.axis_index("s")
        wid = sid * NC + cid

        zeros16 = jnp.zeros((16,), jnp.float32)
        for r in range(ZR):
            for c in range(D // 16):
                z_v[r, pl.ds(c * 16, 16)] = zeros16

        def zero_body(j, carry):
            pltpu.sync_copy(z_v, acc.at[pl.ds(sid * ROWS_PER_TILE + j * ZR, ZR)])
            return carry

        lax.fori_loop(0, ROWS_PER_TILE // ZR, zero_body, 0)
        plsc.subcore_barrier()

        pltpu.sync_copy(w_hbm, w_v)
        w0 = w_v[0]
        w1 = w_v[1]
        dummy16 = jnp.full((16,), DUMMY, jnp.int32)
        base = wid * CHUNKS

        def eload(c, ed_ref, ef_ref, isem, fsem):
            pltpu.async_copy(edi_hbm.at[base + c], ed_ref, isem)
            pltpu.async_copy(edf_hbm.at[base + c], ef_ref, fsem)

        def ewait(ed_ref, ef_ref, isem, fsem):
            pltpu.make_async_copy(edi_hbm.at[0], ed_ref, isem).wait()
            pltpu.make_async_copy(edf_hbm.at[0], ef_ref, fsem).wait()

        def route(ed_ref, ef_ref, idx_ref):
            for v in range(K // 16):
                a0 = ef_ref[pl.ds(v * 16, 16)]
                a1 = ef_ref[pl.ds(K + v * 16, 16)]
                ew = w0 * a0 + w1 * a1
                m = ew != 0.0
                idx_ref[pl.ds(v * 16, 16)] = jnp.where(
                    m, ed_ref[pl.ds(K + v * 16, 16)], dummy16)

        def gather(ed_ref, rows_ref, sem):
            pltpu.async_copy(x_hbm.at[ed_ref.at[pl.ds(0, K)]], rows_ref, sem)

        def gwait(rows_ref, sem):
            pltpu.make_async_copy(x_hbm.at[ed0.at[pl.ds(0, K)]], rows_ref, sem).wait()

        eload(0, ed0, ef0, esem0, fsem0)
        eload(1, ed1, ef1, esem1, fsem1)
        ewait(ed0, ef0, esem0, fsem0)
        gather(ed0, rows0, gsem0)

        def pair_body(j, carry):
            c0 = 2 * j
            ewait(ed1, ef1, esem1, fsem1)
            gather(ed1, rows1, gsem1)
            route(ed0, ef0, idx0)
            gwait(rows0, gsem0)
            pltpu.sync_copy(rows0, acc.at[idx0], add=True)
            eload(c0 + 2, ed0, ef0, esem0, fsem0)
            ewait(ed0, ef0, esem0, fsem0)
            gather(ed0, rows0, gsem0)
            route(ed1, ef1, idx1)
            gwait(rows1, gsem1)
            pltpu.sync_copy(rows1, acc.at[idx1], add=True)
            eload(c0 + 3, ed1, ef1, esem1, fsem1)
            return carry

        lax.fori_loop(0, (CHUNKS - 2) // 2, pair_body, 0)

        ewait(ed1, ef1, esem1, fsem1)
        gather(ed1, rows1, gsem1)
        route(ed0, ef0, idx0)
        gwait(rows0, gsem0)
        pltpu.sync_copy(rows0, acc.at[idx0], add=True)
        route(ed1, ef1, idx1)
        gwait(rows1, gsem1)
        pltpu.sync_copy(rows1, acc.at[idx1], add=True)

        plsc.subcore_barrier()

        pltpu.sync_copy(
            acc.at[pl.ds(sid * ROWS_PER_TILE, ROWS_PER_TILE)],
            out_hbm.at[pl.ds(cid * NP + sid * ROWS_PER_TILE, ROWS_PER_TILE)],
        )

    return sc_kernel(edata_i, edata_f, wvec, x)


def _tc_body(p0_ref, p1_ref, x_ref, wl_ref, wr_ref, b_ref, o_ref):
    agg = p0_ref[...] + p1_ref[...]
    dn = (((1,), (1,)), ((), ()))
    o_ref[...] = (
        lax.dot_general(agg, wl_ref[...], dn, preferred_element_type=jnp.float32)
        + lax.dot_general(x_ref[...], wr_ref[...], dn, preferred_element_type=jnp.float32)
        + b_ref[0:1, :]
    )


def _tc_dense(p0, p1, x, W_l, W_r, b8):
    blk = 1000
    grid = (N // blk,)
    return pl.pallas_call(
        _tc_body,
        grid=grid,
        in_specs=[
            pl.BlockSpec((blk, D), lambda i: (i, 0)),
            pl.BlockSpec((blk, D), lambda i: (i, 0)),
            pl.BlockSpec((blk, D), lambda i: (i, 0)),
            pl.BlockSpec((D, D), lambda i: (0, 0)),
            pl.BlockSpec((D, D), lambda i: (0, 0)),
            pl.BlockSpec((8, D), lambda i: (0, 0)),
        ],
        out_specs=pl.BlockSpec((blk, D), lambda i: (i, 0)),
        out_shape=jax.ShapeDtypeStruct((N, D), jnp.float32),
    )(p0, p1, x, W_l, W_r, b8)


def kernel(x, edge_index, A0, A1, ws_weights, W_l, b_l, W_r):
    src = edge_index[0].astype(jnp.int32)
    dst = edge_index[1].astype(jnp.int32)
    pad = E_PAD - E
    src_p = jnp.concatenate([src, jnp.zeros((pad,), jnp.int32)]).reshape(NW * CHUNKS, K)
    dst_p = jnp.concatenate([dst, jnp.zeros((pad,), jnp.int32)]).reshape(NW * CHUNKS, K)
    a0_p = jnp.concatenate([A0, jnp.zeros((pad,), jnp.float32)]).reshape(NW * CHUNKS, K)
    a1_p = jnp.concatenate([A1, jnp.zeros((pad,), jnp.float32)]).reshape(NW * CHUNKS, K)
    edata_i = jnp.concatenate([src_p, dst_p], axis=1)
    edata_f = jnp.concatenate([a0_p, a1_p], axis=1)
    wvec = jnp.stack([
        jnp.full((16,), ws_weights[0], jnp.float32),
        jnp.full((16,), ws_weights[1], jnp.float32),
    ])

    partials = _sc_scatter(edata_i, edata_f, wvec, x)
    p0 = partials[:N]
    p1 = partials[NP:NP + N]
    b8 = jnp.broadcast_to(b_l.reshape(1, D), (8, D))
    return _tc_dense(p0, p1, x, W_l, W_r, b8)
